# baseline (speedup 1.0000x reference)
import functools

import jax
import jax.numpy as jnp
from jax import lax
from jax.experimental import pallas as pl
from jax.experimental.pallas import tpu as pltpu
from jax.experimental.pallas import tpu_sc as plsc

NN = 10000
NC = 2
NS = 16
LANES = 16
CW = 128


def _mesh():
    return plsc.VectorSubcoreMesh(
        core_axis_name="c", subcore_axis_name="s", num_cores=NC, num_subcores=NS)


def _scatter_rows_sc(rows, x3, bias, n_out):
    m, nc, _ = x3.shape
    kb = 128
    mpt = m // NS
    nb = mpt // kb
    rounds = nc // NC
    rpt = n_out // NS
    zr = 125

    @functools.partial(
        pl.kernel, mesh=_mesh(),
        out_type=jax.ShapeDtypeStruct((n_out, nc, CW), jnp.float32),
        scratch_types=[
            pltpu.VMEM((mpt,), jnp.int32),
            pltpu.VMEM((kb,), jnp.int32),
            pltpu.VMEM((kb, CW), jnp.float32),
            pltpu.VMEM((CW,), jnp.float32),
            pltpu.VMEM((zr, CW), jnp.float32),
            pltpu.VMEM_SHARED((n_out, CW), jnp.float32),
            pltpu.SemaphoreType.DMA,
        ],
    )
    def k(rows_hbm, x_hbm, bias_hbm, out_hbm, rows_v, ridx, gbuf, bvec, bbuf,
          acc, sem):
        core = lax.axis_index("c")
        tid = lax.axis_index("s")
        pltpu.sync_copy(rows_hbm.at[pl.ds(tid * mpt, mpt)], rows_v)
        row0 = tid * rpt
        for r in range(rounds):
            chunk = r * NC + core
            pltpu.sync_copy(bias_hbm.at[pl.ds(chunk * CW, CW)], bvec)

            def initrow(i, _):
                for j in range(CW // LANES):
                    bbuf[i, j * LANES:(j + 1) * LANES] = (
                        bvec[j * LANES:(j + 1) * LANES])
                return 0

            lax.fori_loop(0, zr, initrow, 0)
            for z in range(rpt // zr):
                pltpu.sync_copy(bbuf, acc.at[pl.ds(row0 + z * zr, zr)])
            plsc.subcore_barrier()

            def body(b, _):
                base = tid * mpt + b * kb
                pltpu.async_copy(
                    x_hbm.at[pl.ds(base, kb), chunk], gbuf, sem).wait()
                for j in range(kb // LANES):
                    ridx[j * LANES:(j + 1) * LANES] = (
                        rows_v[pl.ds(b * kb + j * LANES, LANES)])
                pltpu.sync_copy(gbuf, acc.at[ridx], add=True)
                return 0

            lax.fori_loop(0, nb, body, 0)
            plsc.subcore_barrier()
            for z in range(rpt // zr):
                pltpu.sync_copy(acc.at[pl.ds(row0 + z * zr, zr)], bbuf)
                pltpu.sync_copy(
                    bbuf, out_hbm.at[pl.ds(row0 + z * zr, zr), chunk])
            if r + 1 < rounds:
                plsc.subcore_barrier()

    return k(rows, x3, bias)


_KB = 80


def _spmm_sc(raw, vals, x3, bias, n_out, e, r):
    n_in, nc, _ = x3.shape
    kb = _KB
    ept = e // NS
    nb = ept // kb
    rpt = n_out // NS
    zr = 125
    rw = 2 * kb
    x_flat = x3.reshape(n_in * nc, CW)
    assert nb >= 6

    @functools.partial(
        pl.kernel, mesh=_mesh(),
        out_type=jax.ShapeDtypeStruct((n_out, NC, CW), jnp.float32),
        scratch_types=[
            pltpu.VMEM((rw,), jnp.int32),
            pltpu.VMEM((rw,), jnp.int32),
            pltpu.VMEM((rw,), jnp.int32),
            pltpu.VMEM((kb,), jnp.int32),
            pltpu.VMEM((kb,), jnp.int32),
            pltpu.VMEM((kb,), jnp.int32),
            pltpu.VMEM((kb,), jnp.int32),
            pltpu.VMEM((kb,), jnp.int32),
            pltpu.VMEM((kb,), jnp.int32),
            pltpu.VMEM((kb,), jnp.float32),
            pltpu.VMEM((kb,), jnp.float32),
            pltpu.VMEM((kb,), jnp.float32),
            pltpu.VMEM((3, kb, CW), jnp.float32),
            pltpu.VMEM((CW,), jnp.float32),
            pltpu.VMEM((zr, CW), jnp.float32),
            pltpu.VMEM_SHARED((n_out, CW), jnp.float32),
            pltpu.SemaphoreType.DMA((3,)),
            pltpu.SemaphoreType.DMA((3,)),
            pltpu.SemaphoreType.DMA((3,)),
            pltpu.SemaphoreType.DMA((3,)),
        ],
    )
    def k(raw_hbm, vals_hbm, x_hbm, bias_hbm, out_hbm,
          rbuf0, rbuf1, rbuf2, ridx0, ridx1, ridx2, gidx0, gidx1, gidx2,
          vbuf0, vbuf1, vbuf2, gbuf, bvec, bbuf, acc, rsems, vsems, gsems,
          ssems):
        core = lax.axis_index("c")
        tid = lax.axis_index("s")
        row0 = tid * rpt
        rbufs = (rbuf0, rbuf1, rbuf2)
        ridxs = (ridx0, ridx1, ridx2)
        gidxs = (gidx0, gidx1, gidx2)
        vbufs = (vbuf0, vbuf1, vbuf2)

        def issue_raw(b, p):
            pltpu.async_copy(
                raw_hbm.at[pl.ds((tid * nb + b) * rw, rw)], rbufs[p],
                rsems.at[p])
            pltpu.async_copy(
                vals_hbm.at[pl.ds(tid * ept + b * kb, kb)], vbufs[p],
                vsems.at[p])

        def wait_raw(p):
            pltpu.make_async_copy(
                raw_hbm.at[pl.ds(0, rw)], rbufs[p], rsems.at[p]).wait()

        def transform(p, chunk):
            for g in range(kb // LANES):
                sl = pl.ds(g * LANES, LANES)
                ridxs[p][sl] = rbufs[p][pl.ds(g * LANES, LANES)]
                gidxs[p][sl] = (
                    rbufs[p][pl.ds(kb + g * LANES, LANES)] * nc + chunk)

        def issue_gather(p):
            pltpu.async_copy(x_hbm.at[gidxs[p]], gbuf.at[p], gsems.at[p])

        def wait_gather(p):
            pltpu.make_async_copy(
                x_hbm.at[pl.ds(0, kb)], gbuf.at[p], gsems.at[p]).wait()

        def issue_scatter(p):
            pltpu.async_copy(gbuf.at[p], acc.at[ridxs[p]], ssems.at[p],
                             add=True)

        def wait_scatter(p):
            pltpu.make_async_copy(
                x_hbm.at[pl.ds(0, kb)], gbuf.at[p], ssems.at[p]).wait()

        def scale(p):
            pltpu.make_async_copy(
                vals_hbm.at[pl.ds(0, kb)], vbufs[p], vsems.at[p]).wait()
            for g in range(kb // LANES):
                vv = vbufs[p][pl.ds(g * LANES, LANES)]
                for i in range(LANES):
                    v = vv[i]
                    row = g * LANES + i
                    for j in range(CW // LANES):
                        sl = pl.ds(j * LANES, LANES)
                        gbuf[p, row, sl] = gbuf[p, row, sl] * v

        def round_body():
            chunk = r * NC + core
            pltpu.sync_copy(bias_hbm.at[pl.ds(chunk * CW, CW)], bvec)

            def initrow(i, _):
                for j in range(CW // LANES):
                    bbuf[i, j * LANES:(j + 1) * LANES] = (
                        bvec[j * LANES:(j + 1) * LANES])
                return 0

            lax.fori_loop(0, zr, initrow, 0)
            for z in range(rpt // zr):
                pltpu.sync_copy(bbuf, acc.at[pl.ds(row0 + z * zr, zr)])
            plsc.subcore_barrier()

            issue_raw(0, 0)
            issue_raw(1, 1)
            issue_raw(2, 2)
            wait_raw(0)
            transform(0, chunk)
            issue_gather(0)
            wait_raw(1)
            transform(1, chunk)
            issue_gather(1)
            wait_gather(0)
            scale(0)
            issue_scatter(0)
            wait_raw(2)
            transform(2, chunk)
            issue_gather(2)
            issue_raw(3, 0)
            wait_gather(1)
            scale(1)
            issue_scatter(1)
            wait_scatter(0)
            wait_raw(0)
            transform(0, chunk)
            issue_gather(0)
            issue_raw(4, 1)

            def body(it, _):
                for u in range(3):
                    b = 2 + it * 3 + u
                    p = (2 + u) % 3
                    wait_gather(p)
                    scale(p)
                    issue_scatter(p)
                    wait_scatter((p + 2) % 3)
                    wait_raw((p + 2) % 3)
                    transform((p + 2) % 3, chunk)
                    issue_gather((p + 2) % 3)
                    issue_raw(b + 3, p)
                return 0

            lax.fori_loop(0, (nb - 5) // 3, body, 0)
            for b in (nb - 3, nb - 2, nb - 1):
                p = b % 3
                wait_gather(p)
                scale(p)
                issue_scatter(p)
                if b == nb - 3:
                    wait_scatter((p + 2) % 3)
                    wait_raw((p + 2) % 3)
                    transform((p + 2) % 3, chunk)
                    issue_gather((p + 2) % 3)
            wait_scatter((nb - 3) % 3)
            wait_scatter((nb - 2) % 3)
            wait_scatter((nb - 1) % 3)
            plsc.subcore_barrier()
            for z in range(rpt // zr):
                pltpu.sync_copy(acc.at[pl.ds(row0 + z * zr, zr)], bbuf)
                pltpu.sync_copy(
                    bbuf, out_hbm.at[pl.ds(row0 + z * zr, zr), core])

        round_body()

    return k(raw, vals, x_flat, bias)


def _gather_rows_sc(x3, idx):
    n_in, nc, _ = x3.shape
    m = idx.shape[0]
    kb = 128
    mpt = m // NS
    nbg = mpt // kb
    rounds = nc // NC
    x_flat = x3.reshape(n_in * nc, CW)

    @functools.partial(
        pl.kernel, mesh=_mesh(),
        out_type=jax.ShapeDtypeStruct((m, nc, CW), jnp.float32),
        scratch_types=[
            pltpu.VMEM((mpt,), jnp.int32),
            pltpu.VMEM((kb,), jnp.int32),
            pltpu.VMEM((kb, CW), jnp.float32),
            pltpu.SemaphoreType.DMA,
        ],
    )
    def k(x_hbm, idx_hbm, out_hbm, idx_v, gidx, gbuf, sem):
        core = lax.axis_index("c")
        tid = lax.axis_index("s")
        pltpu.sync_copy(idx_hbm.at[pl.ds(tid * mpt, mpt)], idx_v)
        for r in range(rounds):
            chunk = r * NC + core

            def body(g, _):
                for j in range(kb // LANES):
                    sl = pl.ds(j * LANES, LANES)
                    gidx[sl] = (
                        idx_v[pl.ds(g * kb + j * LANES, LANES)] * nc + chunk)
                pltpu.async_copy(x_hbm.at[gidx], gbuf, sem).wait()
                pltpu.sync_copy(
                    gbuf, out_hbm.at[pl.ds(tid * mpt + g * kb, kb), chunk])
                return 0

            lax.fori_loop(0, nbg, body, 0)

    return k(x_flat, idx)


def _matmul_tc(x, w, relu, partial=None):
    n, kdim = x.shape
    f = w.shape[1]
    bm = 400

    def body(x_ref, w_ref, *rest):
        if partial is None:
            o_ref = rest[0]
        else:
            p_ref, o_ref = rest
        xb = x_ref[...]
        if relu:
            xb = jnp.maximum(xb, 0.0)
        acc = jnp.dot(xb, w_ref[...], preferred_element_type=jnp.float32)
        if partial is not None:
            acc = acc + p_ref[...]
        o_ref[...] = acc

    in_specs = [
        pl.BlockSpec((bm, kdim), lambda i: (i, 0)),
        pl.BlockSpec((kdim, f), lambda i: (0, 0)),
    ]
    args = [x, w]
    if partial is not None:
        in_specs.append(pl.BlockSpec((bm, f), lambda i: (i, 0)))
        args.append(partial)
    return pl.pallas_call(
        body,
        grid=(n // bm,),
        in_specs=in_specs,
        out_specs=pl.BlockSpec((bm, f), lambda i: (i, 0)),
        out_shape=jax.ShapeDtypeStruct((n, f), jnp.float32),
    )(*args)


def kernel(adj_indices, adj_values, feat, data_x, W1, b1, W2, b2, W3, b3):
    b = feat.shape[0]
    d = feat.shape[2]
    e = adj_values.shape[0]
    rows = adj_indices[0]
    cols = adj_indices[1]
    raw = jnp.stack(
        [rows.reshape(-1, _KB), cols.reshape(-1, _KB)], axis=1).reshape(-1)
    ev_rows = data_x.reshape(b, 7)[:, :2].reshape(2 * b)
    ev = feat[:, :2, :].reshape(2 * b, d // CW, CW)

    hw = NC * CW

    adjm = _scatter_rows_sc(ev_rows, ev, jnp.zeros((d,), jnp.float32), NN)
    y1 = _matmul_tc(adjm.reshape(NN, d), W1, relu=False)
    y1c = y1.reshape(NN, y1.shape[1] // CW, CW)
    z1a = _spmm_sc(raw, adj_values, y1c, b1, NN, e, 0)
    z1b = _spmm_sc(raw, adj_values, y1c, b1, NN, e, 1)
    y2 = _matmul_tc(z1a.reshape(NN, hw), W2[:hw], relu=True)
    y2 = _matmul_tc(z1b.reshape(NN, hw), W2[hw:], relu=True, partial=y2)
    y2c = y2.reshape(NN, y2.shape[1] // CW, CW)
    z2a = _spmm_sc(raw, adj_values, y2c, b2, NN, e, 0)
    z2b = _spmm_sc(raw, adj_values, y2c, b2, NN, e, 1)
    y3 = _matmul_tc(z2a.reshape(NN, hw), W3[:hw], relu=False)
    y3 = _matmul_tc(z2b.reshape(NN, hw), W3[hw:], relu=False, partial=y3)
    y3c = y3.reshape(NN, y3.shape[1] // CW, CW)
    z3 = _spmm_sc(raw, adj_values, y3c, b3, NN, e, 0)
    g = _gather_rows_sc(z3, ev_rows).reshape(b, 2, d)
    out = feat.at[:, 0, :].set(g[:, 1])
    out = out.at[:, 1, :].set(g[:, 0])
    return out


# speedup vs baseline: 1.0720x; 1.0720x over previous
import functools

import jax
import jax.numpy as jnp
from jax import lax
from jax.experimental import pallas as pl
from jax.experimental.pallas import tpu as pltpu
from jax.experimental.pallas import tpu_sc as plsc

NN = 10000
NC = 2
NS = 16
LANES = 16
CW = 128


def _mesh():
    return plsc.VectorSubcoreMesh(
        core_axis_name="c", subcore_axis_name="s", num_cores=NC, num_subcores=NS)


def _scatter_rows_sc(rows, x3, bias, n_out):
    m, nc, _ = x3.shape
    kb = 128
    mpt = m // NS
    nb = mpt // kb
    rounds = nc // NC
    rpt = n_out // NS
    zr = 125

    @functools.partial(
        pl.kernel, mesh=_mesh(),
        out_type=jax.ShapeDtypeStruct((n_out, nc, CW), jnp.float32),
        scratch_types=[
            pltpu.VMEM((mpt,), jnp.int32),
            pltpu.VMEM((kb,), jnp.int32),
            pltpu.VMEM((kb, CW), jnp.float32),
            pltpu.VMEM((CW,), jnp.float32),
            pltpu.VMEM((zr, CW), jnp.float32),
            pltpu.VMEM_SHARED((n_out, CW), jnp.float32),
            pltpu.SemaphoreType.DMA,
        ],
    )
    def k(rows_hbm, x_hbm, bias_hbm, out_hbm, rows_v, ridx, gbuf, bvec, bbuf,
          acc, sem):
        core = lax.axis_index("c")
        tid = lax.axis_index("s")
        pltpu.sync_copy(rows_hbm.at[pl.ds(tid * mpt, mpt)], rows_v)
        row0 = tid * rpt
        for r in range(rounds):
            chunk = r * NC + core
            pltpu.sync_copy(bias_hbm.at[pl.ds(chunk * CW, CW)], bvec)

            def initrow(i, _):
                for j in range(CW // LANES):
                    bbuf[i, j * LANES:(j + 1) * LANES] = (
                        bvec[j * LANES:(j + 1) * LANES])
                return 0

            lax.fori_loop(0, zr, initrow, 0)
            for z in range(rpt // zr):
                pltpu.sync_copy(bbuf, acc.at[pl.ds(row0 + z * zr, zr)])
            plsc.subcore_barrier()

            def body(b, _):
                base = tid * mpt + b * kb
                pltpu.async_copy(
                    x_hbm.at[pl.ds(base, kb), chunk], gbuf, sem).wait()
                for j in range(kb // LANES):
                    ridx[j * LANES:(j + 1) * LANES] = (
                        rows_v[pl.ds(b * kb + j * LANES, LANES)])
                pltpu.sync_copy(gbuf, acc.at[ridx], add=True)
                return 0

            lax.fori_loop(0, nb, body, 0)
            plsc.subcore_barrier()
            for z in range(rpt // zr):
                pltpu.sync_copy(acc.at[pl.ds(row0 + z * zr, zr)], bbuf)
                pltpu.sync_copy(
                    bbuf, out_hbm.at[pl.ds(row0 + z * zr, zr), chunk])
            if r + 1 < rounds:
                plsc.subcore_barrier()

    return k(rows, x3, bias)


_KB = 80


def _spmm_sc(raw, vals, x3, bias, n_out, e):
    n_in, nc, _ = x3.shape
    kb = _KB
    ept = e // NS
    nb = ept // kb
    rounds = nc // NC
    rpt = n_out // NS
    zr = 125
    rw = 2 * kb
    x_flat = x3.reshape(n_in * nc, CW)
    assert nb >= 6

    @functools.partial(
        pl.kernel, mesh=_mesh(),
        out_type=jax.ShapeDtypeStruct((n_out, nc, CW), jnp.float32),
        scratch_types=[
            pltpu.VMEM((rw,), jnp.int32),
            pltpu.VMEM((rw,), jnp.int32),
            pltpu.VMEM((rw,), jnp.int32),
            pltpu.VMEM((kb,), jnp.int32),
            pltpu.VMEM((kb,), jnp.int32),
            pltpu.VMEM((kb,), jnp.int32),
            pltpu.VMEM((kb,), jnp.int32),
            pltpu.VMEM((kb,), jnp.int32),
            pltpu.VMEM((kb,), jnp.int32),
            pltpu.VMEM((kb,), jnp.float32),
            pltpu.VMEM((kb,), jnp.float32),
            pltpu.VMEM((kb,), jnp.float32),
            pltpu.VMEM((3, kb, CW), jnp.float32),
            pltpu.VMEM((CW,), jnp.float32),
            pltpu.VMEM((zr, CW), jnp.float32),
            pltpu.VMEM_SHARED((n_out, CW), jnp.float32),
            pltpu.SemaphoreType.DMA((3,)),
            pltpu.SemaphoreType.DMA((3,)),
            pltpu.SemaphoreType.DMA((3,)),
            pltpu.SemaphoreType.DMA((3,)),
        ],
    )
    def k(raw_hbm, vals_hbm, x_hbm, bias_hbm, out_hbm,
          rbuf0, rbuf1, rbuf2, ridx0, ridx1, ridx2, gidx0, gidx1, gidx2,
          vbuf0, vbuf1, vbuf2, gbuf, bvec, bbuf, acc, rsems, vsems, gsems,
          ssems):
        core = lax.axis_index("c")
        tid = lax.axis_index("s")
        row0 = tid * rpt
        rbufs = (rbuf0, rbuf1, rbuf2)
        ridxs = (ridx0, ridx1, ridx2)
        gidxs = (gidx0, gidx1, gidx2)
        vbufs = (vbuf0, vbuf1, vbuf2)

        def issue_raw(b, p):
            pltpu.async_copy(
                raw_hbm.at[pl.ds((tid * nb + b) * rw, rw)], rbufs[p],
                rsems.at[p])
            pltpu.async_copy(
                vals_hbm.at[pl.ds(tid * ept + b * kb, kb)], vbufs[p],
                vsems.at[p])

        def wait_raw(p):
            pltpu.make_async_copy(
                raw_hbm.at[pl.ds(0, rw)], rbufs[p], rsems.at[p]).wait()

        def transform(p, chunk):
            for g in range(kb // LANES):
                sl = pl.ds(g * LANES, LANES)
                ridxs[p][sl] = rbufs[p][pl.ds(g * LANES, LANES)]
                gidxs[p][sl] = (
                    rbufs[p][pl.ds(kb + g * LANES, LANES)] * nc + chunk)

        def issue_gather(p):
            pltpu.async_copy(x_hbm.at[gidxs[p]], gbuf.at[p], gsems.at[p])

        def wait_gather(p):
            pltpu.make_async_copy(
                x_hbm.at[pl.ds(0, kb)], gbuf.at[p], gsems.at[p]).wait()

        def issue_scatter(p):
            pltpu.async_copy(gbuf.at[p], acc.at[ridxs[p]], ssems.at[p],
                             add=True)

        def wait_scatter(p):
            pltpu.make_async_copy(
                x_hbm.at[pl.ds(0, kb)], gbuf.at[p], ssems.at[p]).wait()

        def scale(p):
            pltpu.make_async_copy(
                vals_hbm.at[pl.ds(0, kb)], vbufs[p], vsems.at[p]).wait()
            for g in range(kb // LANES):
                vv = vbufs[p][pl.ds(g * LANES, LANES)]
                for i in range(LANES):
                    v = vv[i]
                    row = g * LANES + i
                    for j in range(CW // LANES):
                        sl = pl.ds(j * LANES, LANES)
                        gbuf[p, row, sl] = gbuf[p, row, sl] * v

        def round_body(r, _):
            chunk = r * NC + core
            pltpu.sync_copy(bias_hbm.at[pl.ds(chunk * CW, CW)], bvec)

            def initrow(i, _):
                for j in range(CW // LANES):
                    bbuf[i, j * LANES:(j + 1) * LANES] = (
                        bvec[j * LANES:(j + 1) * LANES])
                return 0

            lax.fori_loop(0, zr, initrow, 0)
            for z in range(rpt // zr):
                pltpu.sync_copy(bbuf, acc.at[pl.ds(row0 + z * zr, zr)])
            plsc.subcore_barrier()

            issue_raw(0, 0)
            issue_raw(1, 1)
            issue_raw(2, 2)
            wait_raw(0)
            transform(0, chunk)
            issue_gather(0)
            wait_raw(1)
            transform(1, chunk)
            issue_gather(1)
            wait_gather(0)
            scale(0)
            issue_scatter(0)
            wait_raw(2)
            transform(2, chunk)
            issue_gather(2)
            issue_raw(3, 0)
            wait_gather(1)
            scale(1)
            issue_scatter(1)
            wait_scatter(0)
            wait_raw(0)
            transform(0, chunk)
            issue_gather(0)
            issue_raw(4, 1)

            def body(it, _):
                for u in range(3):
                    b = 2 + it * 3 + u
                    p = (2 + u) % 3
                    wait_gather(p)
                    scale(p)
                    issue_scatter(p)
                    wait_scatter((p + 2) % 3)
                    wait_raw((p + 2) % 3)
                    transform((p + 2) % 3, chunk)
                    issue_gather((p + 2) % 3)
                    issue_raw(b + 3, p)
                return 0

            lax.fori_loop(0, (nb - 5) // 3, body, 0)
            for b in (nb - 3, nb - 2, nb - 1):
                p = b % 3
                wait_gather(p)
                scale(p)
                issue_scatter(p)
                if b == nb - 3:
                    wait_scatter((p + 2) % 3)
                    wait_raw((p + 2) % 3)
                    transform((p + 2) % 3, chunk)
                    issue_gather((p + 2) % 3)
            wait_scatter((nb - 3) % 3)
            wait_scatter((nb - 2) % 3)
            wait_scatter((nb - 1) % 3)
            plsc.subcore_barrier()
            for z in range(rpt // zr):
                pltpu.sync_copy(acc.at[pl.ds(row0 + z * zr, zr)], bbuf)
                pltpu.sync_copy(
                    bbuf, out_hbm.at[pl.ds(row0 + z * zr, zr), chunk])
            plsc.subcore_barrier()
            return 0

        lax.fori_loop(0, rounds, round_body, 0)

    return k(raw, vals, x_flat, bias)


def _gather_rows_sc(x3, idx):
    n_in, nc, _ = x3.shape
    m = idx.shape[0]
    kb = 128
    mpt = m // NS
    nbg = mpt // kb
    rounds = nc // NC
    x_flat = x3.reshape(n_in * nc, CW)

    @functools.partial(
        pl.kernel, mesh=_mesh(),
        out_type=jax.ShapeDtypeStruct((m, nc, CW), jnp.float32),
        scratch_types=[
            pltpu.VMEM((mpt,), jnp.int32),
            pltpu.VMEM((kb,), jnp.int32),
            pltpu.VMEM((kb, CW), jnp.float32),
            pltpu.SemaphoreType.DMA,
        ],
    )
    def k(x_hbm, idx_hbm, out_hbm, idx_v, gidx, gbuf, sem):
        core = lax.axis_index("c")
        tid = lax.axis_index("s")
        pltpu.sync_copy(idx_hbm.at[pl.ds(tid * mpt, mpt)], idx_v)
        for r in range(rounds):
            chunk = r * NC + core

            def body(g, _):
                for j in range(kb // LANES):
                    sl = pl.ds(j * LANES, LANES)
                    gidx[sl] = (
                        idx_v[pl.ds(g * kb + j * LANES, LANES)] * nc + chunk)
                pltpu.async_copy(x_hbm.at[gidx], gbuf, sem).wait()
                pltpu.sync_copy(
                    gbuf, out_hbm.at[pl.ds(tid * mpt + g * kb, kb), chunk])
                return 0

            lax.fori_loop(0, nbg, body, 0)

    return k(x_flat, idx)


def _matmul_tc(x, w, relu):
    n, kdim = x.shape
    f = w.shape[1]
    bm = 400

    def body(x_ref, w_ref, o_ref):
        xb = x_ref[...]
        if relu:
            xb = jnp.maximum(xb, 0.0)
        o_ref[...] = jnp.dot(xb, w_ref[...],
                             preferred_element_type=jnp.float32)

    return pl.pallas_call(
        body,
        grid=(n // bm,),
        in_specs=[
            pl.BlockSpec((bm, kdim), lambda i: (i, 0)),
            pl.BlockSpec((kdim, f), lambda i: (0, 0)),
        ],
        out_specs=pl.BlockSpec((bm, f), lambda i: (i, 0)),
        out_shape=jax.ShapeDtypeStruct((n, f), jnp.float32),
    )(x, w)


def kernel(adj_indices, adj_values, feat, data_x, W1, b1, W2, b2, W3, b3):
    b = feat.shape[0]
    d = feat.shape[2]
    e = adj_values.shape[0]
    rows = adj_indices[0]
    cols = adj_indices[1]
    raw = jnp.stack(
        [rows.reshape(-1, _KB), cols.reshape(-1, _KB)], axis=1).reshape(-1)
    ev_rows = data_x.reshape(b, 7)[:, :2].reshape(2 * b)
    ev = feat[:, :2, :].reshape(2 * b, d // CW, CW)

    adjm = _scatter_rows_sc(ev_rows, ev, jnp.zeros((d,), jnp.float32), NN)
    y1 = _matmul_tc(adjm.reshape(NN, d), W1, relu=False)
    z1 = _spmm_sc(raw, adj_values, y1.reshape(NN, y1.shape[1] // CW, CW), b1, NN, e)
    y2 = _matmul_tc(z1.reshape(NN, -1), W2, relu=True)
    z2 = _spmm_sc(raw, adj_values, y2.reshape(NN, y2.shape[1] // CW, CW), b2, NN, e)
    y3 = _matmul_tc(z2.reshape(NN, -1), W3, relu=False)
    z3 = _spmm_sc(raw, adj_values, y3.reshape(NN, y3.shape[1] // CW, CW), b3, NN, e)
    g = _gather_rows_sc(z3, ev_rows).reshape(b, 2, d)
    return jnp.concatenate([g[:, 1:2], g[:, 0:1], feat[:, 2:]], axis=1)
